# baseline (speedup 1.0000x reference)
import functools

import jax
import jax.numpy as jnp
from jax.experimental import pallas as pl
from jax.experimental.pallas import tpu as pltpu

TOKENS = 8192
EMB = 2048
E = 16
BEST_K = 2
BLK = 512


def _router_kernel(x_ref, wbt_ref, bb_ref, out_ref, idx_ref):
    logits = jnp.dot(x_ref[...], wbt_ref[...],
                     preferred_element_type=jnp.float32) + bb_ref[...]
    lane = jax.lax.broadcasted_iota(jnp.int32, logits.shape, 1)
    m1 = jnp.max(logits, axis=-1, keepdims=True)
    i1 = jnp.min(jnp.where(logits == m1, lane, E), axis=-1, keepdims=True)
    masked = jnp.where(lane == i1, -jnp.inf, logits)
    m2 = jnp.max(masked, axis=-1, keepdims=True)
    i2 = jnp.min(jnp.where(masked == m2, lane, E), axis=-1, keepdims=True)
    e2 = jnp.exp(m2 - m1)
    denom = 1.0 + e2
    p1 = 1.0 / denom
    p2 = e2 / denom
    out_ref[...] = jnp.where(lane == i1, p1,
                             jnp.where(lane == i2, p2, 0.0))
    idx_ref[...] = jnp.concatenate([i1, i2], axis=-1)


@functools.partial(jax.jit, static_argnames=())
def kernel(x, Wb, bb, Wn, bn):
    del Wn, bn
    wbt = Wb.T
    bb2 = bb.reshape(1, E)
    grid = (TOKENS // BLK,)
    out, idxs = pl.pallas_call(
        _router_kernel,
        grid=grid,
        in_specs=[
            pl.BlockSpec((BLK, EMB), lambda i: (i, 0)),
            pl.BlockSpec((EMB, E), lambda i: (0, 0)),
            pl.BlockSpec((1, E), lambda i: (0, 0)),
        ],
        out_specs=[
            pl.BlockSpec((BLK, E), lambda i: (i, 0)),
            pl.BlockSpec((BLK, BEST_K), lambda i: (i, 0)),
        ],
        out_shape=[
            jax.ShapeDtypeStruct((TOKENS, E), jnp.float32),
            jax.ShapeDtypeStruct((TOKENS, BEST_K), jnp.int32),
        ],
    )(x, wbt, bb2)
    return (out, idxs)


# speedup vs baseline: 1.1488x; 1.1488x over previous
import functools

import jax
import jax.numpy as jnp
from jax.experimental import pallas as pl
from jax.experimental.pallas import tpu as pltpu

TOKENS = 8192
EMB = 2048
E = 16
BEST_K = 2
BLK = 2048


def _router_kernel(x_ref, wbt_ref, bb_ref, out_ref, idx_ref):
    logits = jnp.dot(x_ref[...], wbt_ref[...],
                     preferred_element_type=jnp.float32) + bb_ref[...]
    lane = jax.lax.broadcasted_iota(jnp.int32, logits.shape, 1)
    m1 = jnp.max(logits, axis=-1, keepdims=True)
    i1 = jnp.min(jnp.where(logits == m1, lane, E), axis=-1, keepdims=True)
    masked = jnp.where(lane == i1, -jnp.inf, logits)
    m2 = jnp.max(masked, axis=-1, keepdims=True)
    i2 = jnp.min(jnp.where(masked == m2, lane, E), axis=-1, keepdims=True)
    e2 = jnp.exp(m2 - m1)
    denom = 1.0 + e2
    p1 = 1.0 / denom
    p2 = e2 / denom
    out_ref[...] = jnp.where(lane == i1, p1,
                             jnp.where(lane == i2, p2, 0.0))
    idx_ref[...] = jnp.concatenate([i1, i2], axis=-1)


@functools.partial(jax.jit, static_argnames=())
def kernel(x, Wb, bb, Wn, bn):
    del Wn, bn
    wbt = Wb.T
    bb2 = bb.reshape(1, E)
    grid = (TOKENS // BLK,)
    out, idxs = pl.pallas_call(
        _router_kernel,
        grid=grid,
        in_specs=[
            pl.BlockSpec((BLK, EMB), lambda i: (i, 0)),
            pl.BlockSpec((EMB, E), lambda i: (0, 0)),
            pl.BlockSpec((1, E), lambda i: (0, 0)),
        ],
        out_specs=[
            pl.BlockSpec((BLK, E), lambda i: (i, 0)),
            pl.BlockSpec((BLK, BEST_K), lambda i: (i, 0)),
        ],
        out_shape=[
            jax.ShapeDtypeStruct((TOKENS, E), jnp.float32),
            jax.ShapeDtypeStruct((TOKENS, BEST_K), jnp.int32),
        ],
    )(x, wbt, bb2)
    return (out, idxs)
